# Initial kernel scaffold; baseline (speedup 1.0000x reference)
import jax
import jax.numpy as jnp
from jax.experimental import pallas as pl


def kernel(x, token_table, pos_table):
    raise NotImplementedError("write your pallas kernel here")


# speedup vs baseline: 1.1235x; 1.1235x over previous
import functools

import jax
import jax.numpy as jnp
from jax import lax
from jax.experimental import pallas as pl
from jax.experimental.pallas import tpu as pltpu
from jax.experimental.pallas import tpu_sc as plsc

T = 200
D = 32
NC = 2
NS = 16
NW = NC * NS


def _body(x_hbm, tok_hbm, pos_hbm, out_hbm, idx_v, pos_v, rows_v, sem):
    rows_pw = idx_v.shape[0]
    chunks = rows_pw // T
    wid = lax.axis_index("s") * NC + lax.axis_index("c")
    base = wid * rows_pw

    pltpu.sync_copy(pos_hbm, pos_v)
    pltpu.sync_copy(x_hbm.at[pl.ds(base, rows_pw)], idx_v)

    def chunk_body(c, carry):
        cb = c * T
        g0 = pltpu.async_copy(
            tok_hbm.at[idx_v.at[pl.ds(cb, 128)]], rows_v.at[pl.ds(0, 128)], sem)
        g1 = pltpu.async_copy(
            tok_hbm.at[idx_v.at[pl.ds(cb + 128, 72)]], rows_v.at[pl.ds(128, 72)],
            sem)
        g0.wait()
        g1.wait()

        def add_body(r, carry2):
            rows_v[r, pl.ds(0, 16)] = rows_v[r, pl.ds(0, 16)] + pos_v[r, pl.ds(0, 16)]
            rows_v[r, pl.ds(16, 16)] = rows_v[r, pl.ds(16, 16)] + pos_v[r, pl.ds(16, 16)]
            return carry2

        lax.fori_loop(0, T, add_body, 0, unroll=4)
        pltpu.sync_copy(rows_v, out_hbm.at[pl.ds(base + cb, T)])
        return carry

    lax.fori_loop(0, chunks, chunk_body, 0)


def kernel(x, token_table, pos_table):
    batch, maxlen = x.shape
    n = batch * maxlen
    rows_pw = n // NW

    mesh = plsc.VectorSubcoreMesh(core_axis_name="c", subcore_axis_name="s")
    fn = pl.kernel(
        _body,
        out_type=jax.ShapeDtypeStruct((n, D), jnp.float32),
        mesh=mesh,
        scratch_types=[
            pltpu.VMEM((rows_pw,), jnp.int32),
            pltpu.VMEM((T, D), jnp.float32),
            pltpu.VMEM((T, D), jnp.float32),
            pltpu.SemaphoreType.DMA,
        ],
        compiler_params=pltpu.CompilerParams(use_tc_tiling_on_sc=False),
    )
    out = fn(x.reshape(n).astype(jnp.int32), token_table, pos_table)
    return out.reshape(batch, maxlen, D)


# speedup vs baseline: 1.4124x; 1.2571x over previous
import functools

import jax
import jax.numpy as jnp
from jax import lax
from jax.experimental import pallas as pl
from jax.experimental.pallas import tpu as pltpu
from jax.experimental.pallas import tpu_sc as plsc

T = 200
D = 32
NC = 2
NS = 16
NW = NC * NS
NBUF = 8
LEAD = 4


def _body(x_hbm, tok_hbm, pos_hbm, out_hbm, idx_v, pos_v, rows, *sems):
    sem_g = sems[:NBUF]
    sem_w = sems[NBUF:]
    rows_pw = idx_v.shape[0]
    chunks = rows_pw // T
    wid = lax.axis_index("s") * NC + lax.axis_index("c")
    base = wid * rows_pw

    pltpu.sync_copy(pos_hbm, pos_v)
    pltpu.sync_copy(x_hbm.at[pl.ds(base, rows_pw)], idx_v)

    def issue_gather(c, b):
        cb = c * T
        pltpu.async_copy(
            tok_hbm.at[idx_v.at[pl.ds(cb, 128)]],
            rows.at[b, pl.ds(0, 128)], sem_g[b])
        pltpu.async_copy(
            tok_hbm.at[idx_v.at[pl.ds(cb + 128, 72)]],
            rows.at[b, pl.ds(128, 72)], sem_g[b])

    def wait_gather(b):
        pltpu.make_async_copy(
            tok_hbm.at[pl.ds(0, 128)], rows.at[b, pl.ds(0, 128)], sem_g[b]).wait()
        pltpu.make_async_copy(
            tok_hbm.at[pl.ds(0, 72)], rows.at[b, pl.ds(128, 72)], sem_g[b]).wait()

    def issue_wb(c, b):
        pltpu.async_copy(rows.at[b], out_hbm.at[pl.ds(base + c * T, T)], sem_w[b])

    def wait_wb(b):
        pltpu.make_async_copy(
            rows.at[b], out_hbm.at[pl.ds(0, T)], sem_w[b]).wait()

    for b in range(LEAD):
        issue_gather(b, b)

    def slot(c, b):
        c_pf = c + LEAD

        @pl.when(c_pf < chunks)
        def _():
            @pl.when(c_pf >= NBUF)
            def _():
                wait_wb((b + LEAD) % NBUF)
            issue_gather(c_pf, (b + LEAD) % NBUF)

        wait_gather(b)

        def add_body(r, carry):
            rows[b, r, pl.ds(0, 16)] = rows[b, r, pl.ds(0, 16)] + pos_v[r, pl.ds(0, 16)]
            rows[b, r, pl.ds(16, 16)] = rows[b, r, pl.ds(16, 16)] + pos_v[r, pl.ds(16, 16)]
            return carry

        lax.fori_loop(0, T, add_body, 0, unroll=4)
        issue_wb(c, b)

    def ring_body(i, carry):
        c0 = i * NBUF
        for b in range(NBUF):
            slot(c0 + b, b)
        return carry

    lax.fori_loop(0, chunks // NBUF, ring_body, 0)

    for b in range(NBUF):
        wait_wb(b)


def kernel(x, token_table, pos_table):
    batch, maxlen = x.shape
    n = batch * maxlen
    rows_pw = n // NW

    mesh = plsc.VectorSubcoreMesh(core_axis_name="c", subcore_axis_name="s")
    fn = pl.kernel(
        _body,
        out_type=jax.ShapeDtypeStruct((n, D), jnp.float32),
        mesh=mesh,
        scratch_types=[
            pltpu.VMEM((rows_pw,), jnp.int32),
            pltpu.VMEM((T, D), jnp.float32),
            pltpu.VMEM((NBUF, T, D), jnp.float32),
        ] + [pltpu.SemaphoreType.DMA] * (2 * NBUF),
        compiler_params=pltpu.CompilerParams(use_tc_tiling_on_sc=False),
    )
    out = fn(x.reshape(n).astype(jnp.int32), token_table, pos_table)
    return out.reshape(batch, maxlen, D)


# speedup vs baseline: 1.4156x; 1.0022x over previous
import functools

import jax
import jax.numpy as jnp
from jax import lax
from jax.experimental import pallas as pl
from jax.experimental.pallas import tpu as pltpu
from jax.experimental.pallas import tpu_sc as plsc

T = 200
D = 32
NC = 2
NS = 16
NW = NC * NS
NBUF = 8
LEAD = 4


def _body(x_hbm, tok_hbm, pos_hbm, out_hbm, idx_v, pos_v, rows, *sems):
    sem_g = sems[:NBUF]
    sem_w = sems[NBUF:]
    chunks = idx_v.shape[0]
    wid = lax.axis_index("s") * NC + lax.axis_index("c")
    base = wid * chunks

    pltpu.sync_copy(pos_hbm, pos_v)
    pltpu.sync_copy(x_hbm.at[pl.ds(base, chunks), :], idx_v)

    def issue_gather(c, b):
        pltpu.async_copy(
            tok_hbm.at[idx_v.at[c, pl.ds(0, 128)]],
            rows.at[b, pl.ds(0, 128)], sem_g[b])
        pltpu.async_copy(
            tok_hbm.at[idx_v.at[c, pl.ds(128, 72)]],
            rows.at[b, pl.ds(128, 72)], sem_g[b])

    def wait_gather(b):
        pltpu.make_async_copy(
            tok_hbm.at[pl.ds(0, 128)], rows.at[b, pl.ds(0, 128)], sem_g[b]).wait()
        pltpu.make_async_copy(
            tok_hbm.at[pl.ds(0, 72)], rows.at[b, pl.ds(128, 72)], sem_g[b]).wait()

    def issue_wb(c, b):
        pltpu.async_copy(rows.at[b], out_hbm.at[base + c], sem_w[b])

    def wait_wb(b):
        pltpu.make_async_copy(rows.at[b], out_hbm.at[0], sem_w[b]).wait()

    for b in range(LEAD):
        issue_gather(b, b)

    def slot(c, b):
        c_pf = c + LEAD

        @pl.when(c_pf < chunks)
        def _():
            @pl.when(c_pf >= NBUF)
            def _():
                wait_wb((b + LEAD) % NBUF)
            issue_gather(c_pf, (b + LEAD) % NBUF)

        wait_gather(b)

        def add_body(r, carry):
            rows[b, r, pl.ds(0, 16)] = rows[b, r, pl.ds(0, 16)] + pos_v[r, pl.ds(0, 16)]
            rows[b, r, pl.ds(16, 16)] = rows[b, r, pl.ds(16, 16)] + pos_v[r, pl.ds(16, 16)]
            return carry

        lax.fori_loop(0, T, add_body, 0, unroll=4)
        issue_wb(c, b)

    def ring_body(i, carry):
        c0 = i * NBUF
        for b in range(NBUF):
            slot(c0 + b, b)
        return carry

    lax.fori_loop(0, chunks // NBUF, ring_body, 0)

    for b in range(NBUF):
        wait_wb(b)


def kernel(x, token_table, pos_table):
    batch, maxlen = x.shape
    rows_pw = batch // NW

    mesh = plsc.VectorSubcoreMesh(core_axis_name="c", subcore_axis_name="s")
    fn = pl.kernel(
        _body,
        out_type=jax.ShapeDtypeStruct((batch, maxlen, D), jnp.float32),
        mesh=mesh,
        scratch_types=[
            pltpu.VMEM((rows_pw, T), jnp.int32),
            pltpu.VMEM((T, D), jnp.float32),
            pltpu.VMEM((NBUF, T, D), jnp.float32),
        ] + [pltpu.SemaphoreType.DMA] * (2 * NBUF),
        compiler_params=pltpu.CompilerParams(use_tc_tiling_on_sc=False),
    )
    return fn(x.astype(jnp.int32), token_table, pos_table)


# speedup vs baseline: 1.4318x; 1.0114x over previous
import functools

import jax
import jax.numpy as jnp
from jax import lax
from jax.experimental import pallas as pl
from jax.experimental.pallas import tpu as pltpu
from jax.experimental.pallas import tpu_sc as plsc

T = 200
D = 32
NC = 2
NS = 16
NW = NC * NS
BPW = 128
NBUF = 8
LEAD = 6
TB = 4
XCOL = 40


def _body(x_hbm, tok_hbm, pos_hbm, out_hbm, idxt, xtmp, pos_v, grows, stg, *sems):
    sem_g = sems[:NBUF]
    sem_w = sems[NBUF:NBUF + 2]
    wid = lax.axis_index("s") * NC + lax.axis_index("c")
    base = wid * BPW

    pltpu.sync_copy(pos_hbm, pos_v)

    lanes = lax.broadcasted_iota(jnp.int32, (16,), 0)
    scat0 = lanes * BPW
    scat1 = (lanes + 16) * BPW

    for s in range(T // XCOL):
        pltpu.sync_copy(
            x_hbm.at[pl.ds(base, BPW), pl.ds(s * XCOL, XCOL)], xtmp)

        def xt_body(p, carry):
            b0 = (p % (BPW // 16)) * 16
            tl = p // (BPW // 16)
            tv = jnp.broadcast_to(tl, (16,)).astype(jnp.int32)
            src = plsc.load_gather(xtmp, [b0 + lanes, tv])
            idxt[s * XCOL + tl, pl.ds(b0, 16)] = src
            return carry

        lax.fori_loop(0, (BPW // 16) * XCOL, xt_body, 0, unroll=4)

    def issue_gather(t, g):
        pltpu.async_copy(tok_hbm.at[idxt.at[t, pl.ds(0, BPW)]], grows.at[g], sem_g[g])

    def wait_gather(g):
        pltpu.make_async_copy(
            tok_hbm.at[pl.ds(0, BPW)], grows.at[g], sem_g[g]).wait()

    def issue_wb(t0, s):
        for tl in range(TB):
            for dr in range(4):
                pltpu.async_copy(
                    stg.at[s, pl.ds((tl * 4 + dr) * 8 * BPW, 8 * BPW)],
                    out_hbm.at[t0 + tl].at[dr].at[wid], sem_w[s])

    def wait_wb(s):
        for _ in range(TB * 4):
            pltpu.make_async_copy(
                stg.at[s, pl.ds(0, 8 * BPW)], out_hbm.at[0].at[0].at[0],
                sem_w[s]).wait()

    for g in range(LEAD):
        issue_gather(g, g)

    def slot(t, g, k, s):
        t_pf = t + LEAD

        @pl.when(t_pf < T)
        def _():
            issue_gather(t_pf, (g + LEAD) % NBUF)

        wait_gather(g)
        p0 = pos_v[t, pl.ds(0, 16)]
        p1 = pos_v[t, pl.ds(16, 16)]
        sbase = k * (D * BPW)
        stg_s = stg.at[s]

        def add_body(b, carry):
            v0 = grows[g, b, pl.ds(0, 16)] + p0
            v1 = grows[g, b, pl.ds(16, 16)] + p1
            plsc.store_scatter(stg_s, [scat0 + (sbase + b)], v0)
            plsc.store_scatter(stg_s, [scat1 + (sbase + b)], v1)
            return carry

        lax.fori_loop(0, BPW, add_body, 0, unroll=4)

    def grp_body(i, carry):
        t0 = i * (2 * TB)
        for half in range(2):
            tb0 = t0 + half * TB

            @pl.when(i > 0)
            def _():
                wait_wb(half)

            for k in range(TB):
                slot(tb0 + k, half * TB + k, k, half)
            issue_wb(tb0, half)
        return carry

    lax.fori_loop(0, T // (2 * TB), grp_body, 0)

    for s in range(2):
        wait_wb(s)


def kernel(x, token_table, pos_table):
    batch, maxlen = x.shape

    mesh = plsc.VectorSubcoreMesh(core_axis_name="c", subcore_axis_name="s")
    fn = pl.kernel(
        _body,
        out_type=jax.ShapeDtypeStruct((T, 4, NW, 8 * BPW), jnp.float32),
        mesh=mesh,
        scratch_types=[
            pltpu.VMEM((T, BPW), jnp.int32),
            pltpu.VMEM((BPW, XCOL), jnp.int32),
            pltpu.VMEM((T, D), jnp.float32),
            pltpu.VMEM((NBUF, BPW, D), jnp.float32),
            pltpu.VMEM((2, TB * D * BPW), jnp.float32),
        ] + [pltpu.SemaphoreType.DMA] * (NBUF + 2),
        compiler_params=pltpu.CompilerParams(
            use_tc_tiling_on_sc=False, needs_layout_passes=False),
    )
    out = fn(x.astype(jnp.int32), token_table, pos_table)
    o5 = out.reshape(T, 4, NW, 8, BPW)
    return o5.transpose(2, 4, 0, 1, 3).reshape(batch, maxlen, D)


# speedup vs baseline: 1.7128x; 1.1963x over previous
import functools

import jax
import jax.numpy as jnp
from jax import lax
from jax.experimental import pallas as pl
from jax.experimental.pallas import tpu as pltpu
from jax.experimental.pallas import tpu_sc as plsc

T = 200
D = 32
NC = 2
NS = 16
NW = NC * NS
BPW = 128
NBUF = 8
LEAD = 6
TB = 4
XCOL = 40


def _body(x_hbm, tok_hbm, pos_hbm, out_hbm, idxt, xtmp, pos_v, grows, stg, skew, *sems):
    sem_g = sems[:NBUF]
    sem_w = sems[NBUF:NBUF + 2]
    wid = lax.axis_index("s") * NC + lax.axis_index("c")
    base = wid * BPW

    pltpu.sync_copy(pos_hbm, pos_v)

    lanes = lax.broadcasted_iota(jnp.int32, (16,), 0)
    lanes33 = lanes * 33

    for s in range(T // XCOL):
        pltpu.sync_copy(
            x_hbm.at[pl.ds(base, BPW), pl.ds(s * XCOL, XCOL)], xtmp)

        def xt_body(p, carry):
            b0 = (p % (BPW // 16)) * 16
            tl = p // (BPW // 16)
            tv = jnp.broadcast_to(tl, (16,)).astype(jnp.int32)
            src = plsc.load_gather(xtmp, [b0 + lanes, tv])
            idxt[s * XCOL + tl, pl.ds(b0, 16)] = src
            return carry

        lax.fori_loop(0, (BPW // 16) * XCOL, xt_body, 0, unroll=4)

    def issue_gather(t, g):
        pltpu.async_copy(tok_hbm.at[idxt.at[t, pl.ds(0, BPW)]], grows.at[g], sem_g[g])

    def wait_gather(g):
        pltpu.make_async_copy(
            tok_hbm.at[pl.ds(0, BPW)], grows.at[g], sem_g[g]).wait()

    def issue_wb(t0, s):
        for tl in range(TB):
            for dr in range(4):
                pltpu.async_copy(
                    stg.at[s, pl.ds((tl * 4 + dr) * 8 * BPW, 8 * BPW)],
                    out_hbm.at[t0 + tl].at[dr].at[wid], sem_w[s])

    def wait_wb(s):
        for _ in range(TB * 4):
            pltpu.make_async_copy(
                stg.at[s, pl.ds(0, 8 * BPW)], out_hbm.at[0].at[0].at[0],
                sem_w[s]).wait()

    for g in range(LEAD):
        issue_gather(g, g)

    def slot(t, g, k, s):
        t_pf = t + LEAD

        @pl.when(t_pf < T)
        def _():
            issue_gather(t_pf, (g + LEAD) % NBUF)

        wait_gather(g)
        p0 = pos_v[t, pl.ds(0, 16)]
        p1 = pos_v[t, pl.ds(16, 16)]
        sbase = k * (D * BPW)
        stg_s = stg.at[s]

        def blk_body(bb, carry):
            b0 = bb * 16

            def s1(j, carry2):
                v0 = grows[g, b0 + j, pl.ds(0, 16)] + p0
                v1 = grows[g, b0 + j, pl.ds(16, 16)] + p1
                skew[pl.ds(j * 33, 16)] = v0
                skew[pl.ds(j * 33 + 16, 16)] = v1
                return carry2

            lax.fori_loop(0, 16, s1, 0, unroll=4)

            def s2(d, carry2):
                v = plsc.load_gather(skew, [lanes33 + d])
                stg_s[pl.ds(sbase + d * BPW + b0, 16)] = v
                return carry2

            lax.fori_loop(0, D, s2, 0, unroll=4)
            return carry

        lax.fori_loop(0, BPW // 16, blk_body, 0)

    def grp_body(i, carry):
        t0 = i * (2 * TB)
        for half in range(2):
            tb0 = t0 + half * TB

            @pl.when(i > 0)
            def _():
                wait_wb(half)

            for k in range(TB):
                slot(tb0 + k, half * TB + k, k, half)
            issue_wb(tb0, half)
        return carry

    lax.fori_loop(0, T // (2 * TB), grp_body, 0)

    for s in range(2):
        wait_wb(s)


def kernel(x, token_table, pos_table):
    batch, maxlen = x.shape

    mesh = plsc.VectorSubcoreMesh(core_axis_name="c", subcore_axis_name="s")
    fn = pl.kernel(
        _body,
        out_type=jax.ShapeDtypeStruct((T, 4, NW, 8 * BPW), jnp.float32),
        mesh=mesh,
        scratch_types=[
            pltpu.VMEM((T, BPW), jnp.int32),
            pltpu.VMEM((BPW, XCOL), jnp.int32),
            pltpu.VMEM((T, D), jnp.float32),
            pltpu.VMEM((NBUF, BPW, D), jnp.float32),
            pltpu.VMEM((2, TB * D * BPW), jnp.float32),
            pltpu.VMEM((16 * 33, ), jnp.float32),
        ] + [pltpu.SemaphoreType.DMA] * (NBUF + 2),
        compiler_params=pltpu.CompilerParams(
            use_tc_tiling_on_sc=False, needs_layout_passes=False),
    )
    out = fn(x.astype(jnp.int32), token_table, pos_table)
    o5 = out.reshape(T, 4, NW, 8, BPW)
    return o5.transpose(2, 4, 0, 1, 3).reshape(batch, maxlen, D)


# speedup vs baseline: 2.1245x; 1.2404x over previous
import functools

import jax
import jax.numpy as jnp
from jax import lax
from jax.experimental import pallas as pl
from jax.experimental.pallas import tpu as pltpu
from jax.experimental.pallas import tpu_sc as plsc

T = 200
D = 32
NC = 2
NS = 16
NW = NC * NS
BPW = 128
NBUF = 8
LEAD = 6
TB = 4
XCOL = 40


def _body(x_hbm, tok_hbm, pos_hbm, out_hbm, idxt, xtmp, pos_v, grows, stg, *sems):
    sem_g = sems[:NBUF]
    sem_w = sems[NBUF:NBUF + 2]
    wid = lax.axis_index("s") * NC + lax.axis_index("c")
    base = wid * BPW

    pltpu.sync_copy(pos_hbm, pos_v)

    lanes = lax.broadcasted_iota(jnp.int32, (16,), 0)
    lanes_hi = lanes + 16

    for s in range(T // XCOL):
        pltpu.sync_copy(
            x_hbm.at[pl.ds(base, BPW), pl.ds(s * XCOL, XCOL)], xtmp)

        def xt_body(p, carry):
            b0 = (p % (BPW // 16)) * 16
            tl = p // (BPW // 16)
            tv = jnp.broadcast_to(tl, (16,)).astype(jnp.int32)
            src = plsc.load_gather(xtmp, [b0 + lanes, tv])
            idxt[s * XCOL + tl, pl.ds(b0, 16)] = src
            return carry

        lax.fori_loop(0, (BPW // 16) * XCOL, xt_body, 0, unroll=4)

    def issue_gather(t, g):
        pltpu.async_copy(tok_hbm.at[idxt.at[t, pl.ds(0, BPW)]], grows.at[g], sem_g[g])

    def wait_gather(g):
        pltpu.make_async_copy(
            tok_hbm.at[pl.ds(0, BPW)], grows.at[g], sem_g[g]).wait()

    def issue_wb(t0, s):
        for tl in range(TB):
            for dr in range(4):
                pltpu.async_copy(
                    stg.at[s, tl, pl.ds(dr * 8, 8), pl.ds(0, BPW)],
                    out_hbm.at[t0 + tl].at[dr].at[wid], sem_w[s])

    def wait_wb(s):
        for _ in range(TB * 4):
            pltpu.make_async_copy(
                stg.at[s, 0, pl.ds(0, 8), pl.ds(0, BPW)],
                out_hbm.at[0].at[0].at[0], sem_w[s]).wait()

    for g in range(LEAD):
        issue_gather(g, g)

    def slot(t, g, k, s):
        t_pf = t + LEAD

        @pl.when(t_pf < T)
        def _():
            issue_gather(t_pf, (g + LEAD) % NBUF)

        wait_gather(g)
        p0 = pos_v[t, pl.ds(0, 16)]
        p1 = pos_v[t, pl.ds(16, 16)]

        stg2 = stg.at[s, k]

        def add_body(b, carry):
            bv = jnp.broadcast_to(b, (16,)).astype(jnp.int32)
            v0 = grows[g, b, pl.ds(0, 16)] + p0
            v1 = grows[g, b, pl.ds(16, 16)] + p1
            plsc.store_scatter(stg2, [lanes, bv], v0)
            plsc.store_scatter(stg2, [lanes_hi, bv], v1)
            return carry

        lax.fori_loop(0, BPW, add_body, 0, unroll=8)

    def grp_body(i, carry):
        t0 = i * (2 * TB)
        for half in range(2):
            tb0 = t0 + half * TB

            @pl.when(i > 0)
            def _():
                wait_wb(half)

            for k in range(TB):
                slot(tb0 + k, half * TB + k, k, half)
            issue_wb(tb0, half)
        return carry

    lax.fori_loop(0, T // (2 * TB), grp_body, 0)

    for s in range(2):
        wait_wb(s)


def kernel(x, token_table, pos_table):
    batch, maxlen = x.shape

    mesh = plsc.VectorSubcoreMesh(core_axis_name="c", subcore_axis_name="s")
    fn = pl.kernel(
        _body,
        out_type=jax.ShapeDtypeStruct((T, 4, NW, 8, BPW), jnp.float32),
        mesh=mesh,
        scratch_types=[
            pltpu.VMEM((T, BPW), jnp.int32),
            pltpu.VMEM((BPW, XCOL), jnp.int32),
            pltpu.VMEM((T, D), jnp.float32),
            pltpu.VMEM((NBUF, BPW, D), jnp.float32),
            pltpu.VMEM((2, TB, D, 129), jnp.float32),
        ] + [pltpu.SemaphoreType.DMA] * (NBUF + 2),
        compiler_params=pltpu.CompilerParams(
            use_tc_tiling_on_sc=False, needs_layout_passes=False),
    )
    o5 = fn(x.astype(jnp.int32), token_table, pos_table)
    return o5.transpose(2, 4, 0, 1, 3).reshape(batch, maxlen, D)
